# baseline (speedup 1.0000x reference)
import functools

import jax
import jax.numpy as jnp
from jax import lax
from jax.experimental import pallas as pl
from jax.experimental.pallas import tpu as pltpu
from jax.experimental.pallas import tpu_sc as plsc

B, S, D, E, H, TOP_K = 1, 2048, 768, 8, 3072, 2

RBLK = 256
T = 256
NBLK = 24
NROWS = NBLK * T
TWIN = 544
ST_ALLOC = NROWS + TWIN
NW = 32
TPW = S // NW
NDUMP = 16
NEG_INF = float("-inf")

@functools.lru_cache(maxsize=1)
def _mesh():
    return plsc.VectorSubcoreMesh(core_axis_name="c", subcore_axis_name="s")


def _wid():
    return lax.axis_index("s") * 2 + lax.axis_index("c")


def _iota16():
    return lax.iota(jnp.int32, 16)


def _lane(vec, e):
    return jnp.sum(jnp.where(_iota16() == e, vec, 0))


def _router_body(x_ref, wr_ref, br_ref, wn_ref, bn_ref, noise_ref,
                 gate_ref, e1_ref, e2_ref, g1_ref, g2_ref):
    xb = x_ref[...]
    lg = jnp.dot(xb, wr_ref[...], preferred_element_type=jnp.float32) + br_ref[...]
    nl = jnp.dot(xb, wn_ref[...], preferred_element_type=jnp.float32) + bn_ref[...]
    sp = jnp.maximum(nl, 0.0) + jnp.log1p(jnp.exp(-jnp.abs(nl)))
    noisy = lg + noise_ref[...] * sp
    lane = lax.broadcasted_iota(jnp.int32, (RBLK, E), 1)
    m1 = jnp.max(noisy, axis=1, keepdims=True)
    top1 = noisy == m1
    n2 = jnp.where(top1, NEG_INF, noisy)
    m2 = jnp.max(n2, axis=1, keepdims=True)
    top2 = n2 == m2
    sel = noisy >= m2
    denom = 1.0 + jnp.exp(m2 - m1)
    gate_ref[...] = jnp.where(sel, jnp.exp(noisy - m1), 0.0) / denom
    e1_ref[...] = jnp.min(jnp.where(top1, lane, E), axis=1, keepdims=True)
    e2_ref[...] = jnp.min(jnp.where(top2, lane, E), axis=1, keepdims=True)
    g1_ref[...] = 1.0 / denom
    g2_ref[...] = jnp.exp(m2 - m1) / denom


def _hist_body(e1_hbm, e2_hbm, hist_hbm, va, vb, hbuf, sem):
    w = _wid()
    pltpu.async_copy(e1_hbm.at[pl.ds(w * TPW, TPW)], va, sem).wait()
    pltpu.async_copy(e2_hbm.at[pl.ds(w * TPW, TPW)], vb, sem).wait()
    iota = _iota16()
    hv = jnp.zeros((16,), jnp.int32)
    for e in range(E):
        cnt = jnp.int32(0)
        for v in range(TPW // 16):
            cnt = cnt + jnp.sum((va[pl.ds(16 * v, 16)] == e).astype(jnp.int32))
            cnt = cnt + jnp.sum((vb[pl.ds(16 * v, 16)] == e).astype(jnp.int32))
        hv = hv + cnt * (iota == e).astype(jnp.int32)
    hbuf[...] = hv
    pltpu.sync_copy(hbuf, hist_hbm.at[pl.ds(16 * w, 16)])


def _place_body(hist_hbm, e1_hbm, e2_hbm, g1_hbm, g2_hbm, x_hbm,
                st_hbm, sg_hbm, xs_hbm, p0_hbm, p1_hbm, bounds_hbm,
                bexp_hbm, bval_hbm, xsmap_hbm,
                hv, va, vb, gbuf, tokbuf, posbuf, mbuf, myx, sem, sem2):
    w = _wid()
    iota = _iota16()
    pltpu.async_copy(hist_hbm, hv, sem).wait()

    def _hrow(i):
        return hv[pl.ds(16 * i, 16)]

    def tot_step(i, acc):
        return acc + hv[pl.ds(16 * i, 16)]
    total = lax.fori_loop(0, NW, tot_step, jnp.zeros((16,), jnp.int32))
    total = jnp.where(iota < E, total, 0)
    padded = ((total + (T - 1)) // T) * T
    pad_off = plsc.cumsum(padded) - padded
    nblocks_end = (pad_off + padded) // T

    prefix = lax.fori_loop(0, w, tot_step, jnp.zeros((16,), jnp.int32))
    base_vec = pad_off + prefix

    c1 = pltpu.async_copy(e1_hbm.at[pl.ds(w * TPW, TPW)], va, sem)
    c2 = pltpu.async_copy(e2_hbm.at[pl.ds(w * TPW, TPW)], vb, sem)
    c3 = pltpu.async_copy(g1_hbm.at[pl.ds(w * TPW, TPW)], gbuf.at[pl.ds(0, TPW)], sem)
    c4 = pltpu.async_copy(g2_hbm.at[pl.ds(w * TPW, TPW)], gbuf.at[pl.ds(TPW, TPW)], sem)
    cx = pltpu.async_copy(x_hbm.at[pl.ds(w * TPW, TPW)], myx, sem2)
    c1.wait(); c2.wait(); c3.wait(); c4.wait()

    base = [_lane(base_vec, e) for e in range(E)]
    for k in range(2):
        src = va if k == 0 else vb
        for v in range(TPW // 16):
            ev = src[pl.ds(16 * v, 16)]
            tokbuf[pl.ds(k * TPW + 16 * v, 16)] = w * TPW + 16 * v + iota
            posv = jnp.zeros((16,), jnp.int32)
            for e in range(E):
                mi = (ev == e).astype(jnp.int32)
                rank = plsc.cumsum(mi) - mi
                posv = jnp.where(mi > 0, base[e] + rank, posv)
                base[e] = base[e] + jnp.sum(mi)
            posbuf[k, pl.ds(16 * v, 16)] = posv

    cx.wait()
    d1 = pltpu.async_copy(tokbuf.at[pl.ds(0, TPW)], st_hbm.at[posbuf.at[0]], sem)
    d2 = pltpu.async_copy(tokbuf.at[pl.ds(TPW, TPW)], st_hbm.at[posbuf.at[1]], sem)
    d3 = pltpu.async_copy(gbuf.at[pl.ds(0, TPW)], sg_hbm.at[posbuf.at[0]], sem)
    d4 = pltpu.async_copy(gbuf.at[pl.ds(TPW, TPW)], sg_hbm.at[posbuf.at[1]], sem)
    d5 = pltpu.async_copy(myx, xs_hbm.at[posbuf.at[0]], sem2)
    d6 = pltpu.async_copy(myx, xs_hbm.at[posbuf.at[1]], sem2)
    pltpu.sync_copy(posbuf.at[0], p0_hbm.at[pl.ds(w * TPW, TPW)])
    pltpu.sync_copy(posbuf.at[1], p1_hbm.at[pl.ds(w * TPW, TPW)])
    d1.wait(); d2.wait(); d3.wait(); d4.wait(); d5.wait(); d6.wait()

    @pl.when(w == 0)
    def _():
        bnd = pad_off
        mbuf[pl.ds(0, 16)] = bnd
        for m in range(1, 17):
            bnd = bnd + _hrow(2 * m - 2) + _hrow(2 * m - 1)
            mbuf[pl.ds(16 * m, 16)] = bnd
        pltpu.sync_copy(mbuf.at[pl.ds(0, 272)], bounds_hbm)

        nb = _lane(nblocks_end, E - 1)
        ends = [_lane(nblocks_end, e) for e in range(E)]
        for v in range((NBLK + 15) // 16):
            jv = jnp.minimum(16 * v + iota, nb - 1)
            bexp = jnp.zeros((16,), jnp.int32)
            for e in range(E):
                bexp = bexp + (jv >= ends[e]).astype(jnp.int32)
            mbuf[pl.ds(288 + 16 * v, 16)] = bexp
            mbuf[pl.ds(352 + 16 * v, 16)] = ((16 * v + iota) < nb).astype(jnp.int32)
            mbuf[pl.ds(416 + 16 * v, 16)] = jv
        pltpu.sync_copy(mbuf.at[pl.ds(288, 32)], bexp_hbm)
        pltpu.sync_copy(mbuf.at[pl.ds(352, 32)], bval_hbm)
        pltpu.sync_copy(mbuf.at[pl.ds(416, 32)], xsmap_hbm)


def _ffn_body(bexp_ref, bval_ref, xsmap_ref,
              xs_ref, w1_ref, b1_ref, w2_ref, b2_ref, g_ref, ys_ref):
    j = pl.program_id(0)

    @pl.when(bval_ref[j] == 1)
    def _():
        hblk = jax.nn.relu(
            jnp.dot(xs_ref[0], w1_ref[0], preferred_element_type=jnp.float32)
            + b1_ref[0])
        ys_ref[0] = (jnp.dot(hblk, w2_ref[0], preferred_element_type=jnp.float32)
                     + b2_ref[0]) * g_ref[0]


def _expand_body(ys_hbm, st_hbm, bounds_hbm, p0_hbm, p1_hbm,
                 dense_hbm, fin_hbm,
                 zbuf, gbuf, bbuf, cbuf, dbuf, tokwin, bnds, pbuf,
                 sem, semb, semz):
    w = _wid()
    iota = _iota16()
    e = w // 4
    q = w % 4

    for r in range(16):
        def zc(c, _):
            zbuf[r, pl.ds(16 * c, 16)] = jnp.zeros((16,), jnp.float32)
            return 0
        lax.fori_loop(0, D // 16, zc, 0)
    row0 = e * S + q * 512
    zcopies = [
        pltpu.async_copy(zbuf, dense_hbm.at[pl.ds(row0 + 16 * i, 16)], semz)
        for i in range(32)
    ]

    cb0 = pltpu.async_copy(bounds_hbm.at[pl.ds(64 * q, 16)], bnds.at[pl.ds(0, 16)], sem)
    cb1 = pltpu.async_copy(bounds_hbm.at[pl.ds(64 * q + 64, 16)], bnds.at[pl.ds(16, 16)], sem)
    cp0 = pltpu.async_copy(p0_hbm.at[pl.ds(w * TPW, TPW)], pbuf.at[pl.ds(0, TPW)], semb)
    cp1 = pltpu.async_copy(p1_hbm.at[pl.ds(w * TPW, TPW)], pbuf.at[pl.ds(TPW, TPW)], semb)
    cb0.wait()
    cb1.wait()
    lo = _lane(bnds[pl.ds(0, 16)], e)
    hi = _lane(bnds[pl.ds(16, 16)], e)
    lo8 = (lo // 8) * 8
    ct = pltpu.async_copy(st_hbm.at[pl.ds(lo8, TWIN)], tokwin, sem)
    for z in zcopies:
        z.wait()
    ct.wait()

    nwin = (hi - lo8 + 15) // 16

    def wvec(i):
        jv = lo8 + 16 * i + iota
        ok = (jv >= lo) & (jv < hi)
        jc = jnp.minimum(jnp.where(ok, jv, lo), NROWS - 1)
        tok = tokwin[pl.ds(16 * i, 16)]
        dest = jnp.where(ok, e * S + tok, 16384 + iota)
        return jc, dest

    def pair(k, _):
        jc0, d0 = wvec(2 * k)
        jc1, d1 = wvec(2 * k + 1)
        g0 = pltpu.async_copy(ys_hbm.at[jc0], gbuf, sem)
        g1 = pltpu.async_copy(ys_hbm.at[jc1], bbuf, semb)
        g0.wait()
        s0 = pltpu.async_copy(gbuf, dense_hbm.at[d0], sem)
        g1.wait()
        s1 = pltpu.async_copy(bbuf, dense_hbm.at[d1], semb)
        s0.wait()
        s1.wait()
        return 0
    lax.fori_loop(0, (nwin + 1) // 2, pair, 0)

    cp0.wait()
    cp1.wait()
    prev = None
    for i in range(TPW // 16):
        ga, gb = (gbuf, bbuf) if i % 2 == 0 else (cbuf, dbuf)
        gA = pltpu.async_copy(ys_hbm.at[pbuf[pl.ds(16 * i, 16)]], ga, sem)
        gB = pltpu.async_copy(ys_hbm.at[pbuf[pl.ds(TPW + 16 * i, 16)]], gb, semb)
        gA.wait()
        gB.wait()
        if prev is not None:
            prev.wait()

        def fc(c, _):
            for r in range(16):
                zbuf[r, pl.ds(16 * c, 16)] = (ga[r, pl.ds(16 * c, 16)]
                                              + gb[r, pl.ds(16 * c, 16)])
            return 0
        lax.fori_loop(0, D // 16, fc, 0)
        prev = pltpu.async_copy(zbuf, fin_hbm.at[pl.ds(w * TPW + 16 * i, 16)], semz)
    prev.wait()


@jax.jit
def kernel(x, noise, Wr, br, Wn, bn, W1, b1, W2, b2):
    x2 = x.reshape(S, D)
    noise2 = noise.reshape(S, E)

    gate, e1, e2, g1, g2 = pl.pallas_call(
        _router_body,
        grid=(S // RBLK,),
        in_specs=[
            pl.BlockSpec((RBLK, D), lambda i: (i, 0)),
            pl.BlockSpec((D, E), lambda i: (0, 0)),
            pl.BlockSpec((E,), lambda i: (0,)),
            pl.BlockSpec((D, E), lambda i: (0, 0)),
            pl.BlockSpec((E,), lambda i: (0,)),
            pl.BlockSpec((RBLK, E), lambda i: (i, 0)),
        ],
        out_specs=[
            pl.BlockSpec((RBLK, E), lambda i: (i, 0)),
            pl.BlockSpec((RBLK, 1), lambda i: (i, 0)),
            pl.BlockSpec((RBLK, 1), lambda i: (i, 0)),
            pl.BlockSpec((RBLK, 1), lambda i: (i, 0)),
            pl.BlockSpec((RBLK, 1), lambda i: (i, 0)),
        ],
        out_shape=[
            jax.ShapeDtypeStruct((S, E), jnp.float32),
            jax.ShapeDtypeStruct((S, 1), jnp.int32),
            jax.ShapeDtypeStruct((S, 1), jnp.int32),
            jax.ShapeDtypeStruct((S, 1), jnp.float32),
            jax.ShapeDtypeStruct((S, 1), jnp.float32),
        ],
    )(x2, Wr, br, Wn, bn, noise2)
    e1f, e2f = e1.reshape(S), e2.reshape(S)
    g1f, g2f = g1.reshape(S), g2.reshape(S)

    hist = pl.kernel(
        _hist_body,
        out_type=jax.ShapeDtypeStruct((NW * 16,), jnp.int32),
        mesh=_mesh(),
        compiler_params=pltpu.CompilerParams(needs_layout_passes=False),
        scratch_types=[
            pltpu.VMEM((TPW,), jnp.int32),
            pltpu.VMEM((TPW,), jnp.int32),
            pltpu.VMEM((16,), jnp.int32),
            pltpu.SemaphoreType.DMA,
        ],
    )(e1f, e2f)

    st, sg, xs, p0, p1, bounds, bexp, bval, xsmap = pl.kernel(
        _place_body,
        out_type=(
            jax.ShapeDtypeStruct((ST_ALLOC,), jnp.int32),
            jax.ShapeDtypeStruct((NROWS,), jnp.float32),
            jax.ShapeDtypeStruct((NROWS, D), jnp.float32),
            jax.ShapeDtypeStruct((S,), jnp.int32),
            jax.ShapeDtypeStruct((S,), jnp.int32),
            jax.ShapeDtypeStruct((272,), jnp.int32),
            jax.ShapeDtypeStruct((32,), jnp.int32),
            jax.ShapeDtypeStruct((32,), jnp.int32),
            jax.ShapeDtypeStruct((32,), jnp.int32),
        ),
        mesh=_mesh(),
        compiler_params=pltpu.CompilerParams(needs_layout_passes=False),
        scratch_types=[
            pltpu.VMEM((NW * 16,), jnp.int32),
            pltpu.VMEM((TPW,), jnp.int32),
            pltpu.VMEM((TPW,), jnp.int32),
            pltpu.VMEM((2 * TPW,), jnp.float32),
            pltpu.VMEM((2 * TPW,), jnp.int32),
            pltpu.VMEM((2, TPW), jnp.int32),
            pltpu.VMEM((448,), jnp.int32),
            pltpu.VMEM((TPW, D), jnp.float32),
            pltpu.SemaphoreType.DMA,
            pltpu.SemaphoreType.DMA,
        ],
    )(hist, e1f, e2f, g1f, g2f, x2)

    ys = pl.pallas_call(
        _ffn_body,
        grid_spec=pltpu.PrefetchScalarGridSpec(
            num_scalar_prefetch=3,
            grid=(NBLK,),
            in_specs=[
                pl.BlockSpec((1, T, D), lambda j, be, bv, xm: (xm[j], 0, 0)),
                pl.BlockSpec((1, D, H), lambda j, be, bv, xm: (be[j], 0, 0)),
                pl.BlockSpec((1, 1, H), lambda j, be, bv, xm: (be[j], 0, 0)),
                pl.BlockSpec((1, H, D), lambda j, be, bv, xm: (be[j], 0, 0)),
                pl.BlockSpec((1, 1, D), lambda j, be, bv, xm: (be[j], 0, 0)),
                pl.BlockSpec((1, T, 1), lambda j, be, bv, xm: (xm[j], 0, 0)),
            ],
            out_specs=pl.BlockSpec((1, T, D), lambda j, be, bv, xm: (j, 0, 0)),
        ),
        out_shape=jax.ShapeDtypeStruct((NBLK, T, D), jnp.float32),
        compiler_params=pltpu.CompilerParams(
            dimension_semantics=("arbitrary",),
        ),
    )(bexp, bval, xsmap,
      xs.reshape(NBLK, T, D), W1, b1.reshape(E, 1, H), W2, b2.reshape(E, 1, D),
      sg.reshape(NBLK, T, 1))

    dense, fin = pl.kernel(
        _expand_body,
        out_type=(
            jax.ShapeDtypeStruct((16384 + NDUMP, D), jnp.float32),
            jax.ShapeDtypeStruct((S, D), jnp.float32),
        ),
        mesh=_mesh(),
        compiler_params=pltpu.CompilerParams(needs_layout_passes=False),
        scratch_types=[
            pltpu.VMEM((16, D), jnp.float32),
            pltpu.VMEM((16, D), jnp.float32),
            pltpu.VMEM((16, D), jnp.float32),
            pltpu.VMEM((16, D), jnp.float32),
            pltpu.VMEM((16, D), jnp.float32),
            pltpu.VMEM((TWIN,), jnp.int32),
            pltpu.VMEM((32,), jnp.int32),
            pltpu.VMEM((2 * TPW,), jnp.int32),
            pltpu.SemaphoreType.DMA,
            pltpu.SemaphoreType.DMA,
            pltpu.SemaphoreType.DMA,
        ],
    )(ys.reshape(NROWS, D), st, bounds, p0, p1)

    return (fin.reshape(B, S, D),
            dense[:16384].reshape(E, B, S, D),
            gate.reshape(B, S, E))


# speedup vs baseline: 1.0646x; 1.0646x over previous
import functools

import jax
import jax.numpy as jnp
from jax import lax
from jax.experimental import pallas as pl
from jax.experimental.pallas import tpu as pltpu
from jax.experimental.pallas import tpu_sc as plsc

B, S, D, E, H, TOP_K = 1, 2048, 768, 8, 3072, 2

RBLK = 256
T = 512
NBLK = 16
NROWS = NBLK * T
TWIN = 544
ST_ALLOC = NROWS + TWIN
NW = 32
TPW = S // NW
NDUMP = 16
NEG_INF = float("-inf")

@functools.lru_cache(maxsize=1)
def _mesh():
    return plsc.VectorSubcoreMesh(core_axis_name="c", subcore_axis_name="s")


def _wid():
    return lax.axis_index("s") * 2 + lax.axis_index("c")


def _iota16():
    return lax.iota(jnp.int32, 16)


def _lane(vec, e):
    return jnp.sum(jnp.where(_iota16() == e, vec, 0))


def _router_body(x_ref, wr_ref, br_ref, wn_ref, bn_ref, noise_ref,
                 gate_ref, e1_ref, e2_ref, g1_ref, g2_ref):
    xb = x_ref[...]
    lg = jnp.dot(xb, wr_ref[...], preferred_element_type=jnp.float32) + br_ref[...]
    nl = jnp.dot(xb, wn_ref[...], preferred_element_type=jnp.float32) + bn_ref[...]
    sp = jnp.maximum(nl, 0.0) + jnp.log1p(jnp.exp(-jnp.abs(nl)))
    noisy = lg + noise_ref[...] * sp
    lane = lax.broadcasted_iota(jnp.int32, (RBLK, E), 1)
    m1 = jnp.max(noisy, axis=1, keepdims=True)
    top1 = noisy == m1
    n2 = jnp.where(top1, NEG_INF, noisy)
    m2 = jnp.max(n2, axis=1, keepdims=True)
    top2 = n2 == m2
    sel = noisy >= m2
    denom = 1.0 + jnp.exp(m2 - m1)
    gate_ref[...] = jnp.where(sel, jnp.exp(noisy - m1), 0.0) / denom
    e1_ref[...] = jnp.min(jnp.where(top1, lane, E), axis=1, keepdims=True)
    e2_ref[...] = jnp.min(jnp.where(top2, lane, E), axis=1, keepdims=True)
    g1_ref[...] = 1.0 / denom
    g2_ref[...] = jnp.exp(m2 - m1) / denom


def _hist_body(e1_hbm, e2_hbm, hist_hbm, va, vb, hbuf, sem):
    w = _wid()
    pltpu.async_copy(e1_hbm.at[pl.ds(w * TPW, TPW)], va, sem).wait()
    pltpu.async_copy(e2_hbm.at[pl.ds(w * TPW, TPW)], vb, sem).wait()
    iota = _iota16()
    hv = jnp.zeros((16,), jnp.int32)
    for e in range(E):
        cnt = jnp.int32(0)
        for v in range(TPW // 16):
            cnt = cnt + jnp.sum((va[pl.ds(16 * v, 16)] == e).astype(jnp.int32))
            cnt = cnt + jnp.sum((vb[pl.ds(16 * v, 16)] == e).astype(jnp.int32))
        hv = hv + cnt * (iota == e).astype(jnp.int32)
    hbuf[...] = hv
    pltpu.sync_copy(hbuf, hist_hbm.at[pl.ds(16 * w, 16)])


def _place_body(hist_hbm, e1_hbm, e2_hbm, g1_hbm, g2_hbm, x_hbm,
                st_hbm, sg_hbm, xs_hbm, p0_hbm, p1_hbm, bounds_hbm,
                bexp_hbm, bval_hbm, xsmap_hbm,
                hv, va, vb, gbuf, tokbuf, posbuf, mbuf, myx, sem, sem2):
    w = _wid()
    iota = _iota16()
    pltpu.async_copy(hist_hbm, hv, sem).wait()

    def _hrow(i):
        return hv[pl.ds(16 * i, 16)]

    def tot_step(i, acc):
        return acc + hv[pl.ds(16 * i, 16)]
    total = lax.fori_loop(0, NW, tot_step, jnp.zeros((16,), jnp.int32))
    total = jnp.where(iota < E, total, 0)
    padded = ((total + (T - 1)) // T) * T
    pad_off = plsc.cumsum(padded) - padded
    nblocks_end = (pad_off + padded) // T

    prefix = lax.fori_loop(0, w, tot_step, jnp.zeros((16,), jnp.int32))
    base_vec = pad_off + prefix

    c1 = pltpu.async_copy(e1_hbm.at[pl.ds(w * TPW, TPW)], va, sem)
    c2 = pltpu.async_copy(e2_hbm.at[pl.ds(w * TPW, TPW)], vb, sem)
    c3 = pltpu.async_copy(g1_hbm.at[pl.ds(w * TPW, TPW)], gbuf.at[pl.ds(0, TPW)], sem)
    c4 = pltpu.async_copy(g2_hbm.at[pl.ds(w * TPW, TPW)], gbuf.at[pl.ds(TPW, TPW)], sem)
    cx = pltpu.async_copy(x_hbm.at[pl.ds(w * TPW, TPW)], myx, sem2)
    c1.wait(); c2.wait(); c3.wait(); c4.wait()

    base = [_lane(base_vec, e) for e in range(E)]
    for k in range(2):
        src = va if k == 0 else vb
        for v in range(TPW // 16):
            ev = src[pl.ds(16 * v, 16)]
            tokbuf[pl.ds(k * TPW + 16 * v, 16)] = w * TPW + 16 * v + iota
            posv = jnp.zeros((16,), jnp.int32)
            for e in range(E):
                mi = (ev == e).astype(jnp.int32)
                rank = plsc.cumsum(mi) - mi
                posv = jnp.where(mi > 0, base[e] + rank, posv)
                base[e] = base[e] + jnp.sum(mi)
            posbuf[k, pl.ds(16 * v, 16)] = posv

    cx.wait()
    d1 = pltpu.async_copy(tokbuf.at[pl.ds(0, TPW)], st_hbm.at[posbuf.at[0]], sem)
    d2 = pltpu.async_copy(tokbuf.at[pl.ds(TPW, TPW)], st_hbm.at[posbuf.at[1]], sem)
    d3 = pltpu.async_copy(gbuf.at[pl.ds(0, TPW)], sg_hbm.at[posbuf.at[0]], sem)
    d4 = pltpu.async_copy(gbuf.at[pl.ds(TPW, TPW)], sg_hbm.at[posbuf.at[1]], sem)
    d5 = pltpu.async_copy(myx, xs_hbm.at[posbuf.at[0]], sem2)
    d6 = pltpu.async_copy(myx, xs_hbm.at[posbuf.at[1]], sem2)
    pltpu.sync_copy(posbuf.at[0], p0_hbm.at[pl.ds(w * TPW, TPW)])
    pltpu.sync_copy(posbuf.at[1], p1_hbm.at[pl.ds(w * TPW, TPW)])
    d1.wait(); d2.wait(); d3.wait(); d4.wait(); d5.wait(); d6.wait()

    @pl.when(w == 0)
    def _():
        bnd = pad_off
        mbuf[pl.ds(0, 16)] = bnd
        for m in range(1, 17):
            bnd = bnd + _hrow(2 * m - 2) + _hrow(2 * m - 1)
            mbuf[pl.ds(16 * m, 16)] = bnd
        pltpu.sync_copy(mbuf.at[pl.ds(0, 272)], bounds_hbm)

        nb = _lane(nblocks_end, E - 1)
        ends = [_lane(nblocks_end, e) for e in range(E)]
        for v in range((NBLK + 15) // 16):
            jv = jnp.minimum(16 * v + iota, nb - 1)
            bexp = jnp.zeros((16,), jnp.int32)
            for e in range(E):
                bexp = bexp + (jv >= ends[e]).astype(jnp.int32)
            mbuf[pl.ds(288 + 16 * v, 16)] = bexp
            mbuf[pl.ds(352 + 16 * v, 16)] = ((16 * v + iota) < nb).astype(jnp.int32)
            mbuf[pl.ds(416 + 16 * v, 16)] = jv
        pltpu.sync_copy(mbuf.at[pl.ds(288, 32)], bexp_hbm)
        pltpu.sync_copy(mbuf.at[pl.ds(352, 32)], bval_hbm)
        pltpu.sync_copy(mbuf.at[pl.ds(416, 32)], xsmap_hbm)


def _ffn_body(bexp_ref, bval_ref, xsmap_ref,
              xs_ref, w1_ref, b1_ref, w2_ref, b2_ref, g_ref, ys_ref):
    j = pl.program_id(0)

    @pl.when(bval_ref[j] == 1)
    def _():
        hblk = jax.nn.relu(
            jnp.dot(xs_ref[0], w1_ref[0], preferred_element_type=jnp.float32)
            + b1_ref[0])
        ys_ref[0] = (jnp.dot(hblk, w2_ref[0], preferred_element_type=jnp.float32)
                     + b2_ref[0]) * g_ref[0]


def _expand_body(ys_hbm, st_hbm, bounds_hbm, p0_hbm, p1_hbm,
                 dense_hbm, fin_hbm,
                 zbuf, gbuf, bbuf, cbuf, dbuf, tokwin, bnds, pbuf,
                 sem, semb, semz):
    w = _wid()
    iota = _iota16()
    e = w // 4
    q = w % 4

    for r in range(16):
        def zc(c, _):
            zbuf[r, pl.ds(16 * c, 16)] = jnp.zeros((16,), jnp.float32)
            return 0
        lax.fori_loop(0, D // 16, zc, 0)
    row0 = e * S + q * 512
    zcopies = [
        pltpu.async_copy(zbuf, dense_hbm.at[pl.ds(row0 + 16 * i, 16)], semz)
        for i in range(32)
    ]

    cb0 = pltpu.async_copy(bounds_hbm.at[pl.ds(64 * q, 16)], bnds.at[pl.ds(0, 16)], sem)
    cb1 = pltpu.async_copy(bounds_hbm.at[pl.ds(64 * q + 64, 16)], bnds.at[pl.ds(16, 16)], sem)
    cp0 = pltpu.async_copy(p0_hbm.at[pl.ds(w * TPW, TPW)], pbuf.at[pl.ds(0, TPW)], semb)
    cp1 = pltpu.async_copy(p1_hbm.at[pl.ds(w * TPW, TPW)], pbuf.at[pl.ds(TPW, TPW)], semb)
    cb0.wait()
    cb1.wait()
    lo = _lane(bnds[pl.ds(0, 16)], e)
    hi = _lane(bnds[pl.ds(16, 16)], e)
    lo8 = (lo // 8) * 8
    ct = pltpu.async_copy(st_hbm.at[pl.ds(lo8, TWIN)], tokwin, sem)
    for z in zcopies:
        z.wait()
    ct.wait()

    nwin = (hi - lo8 + 15) // 16

    def wvec(i):
        jv = lo8 + 16 * i + iota
        ok = (jv >= lo) & (jv < hi)
        jc = jnp.minimum(jnp.where(ok, jv, lo), NROWS - 1)
        tok = tokwin[pl.ds(16 * i, 16)]
        dest = jnp.where(ok, e * S + tok, 16384 + iota)
        return jc, dest

    def pair(k, _):
        jc0, d0 = wvec(2 * k)
        jc1, d1 = wvec(2 * k + 1)
        g0 = pltpu.async_copy(ys_hbm.at[jc0], gbuf, sem)
        g1 = pltpu.async_copy(ys_hbm.at[jc1], bbuf, semb)
        g0.wait()
        s0 = pltpu.async_copy(gbuf, dense_hbm.at[d0], sem)
        g1.wait()
        s1 = pltpu.async_copy(bbuf, dense_hbm.at[d1], semb)
        s0.wait()
        s1.wait()
        return 0
    lax.fori_loop(0, (nwin + 1) // 2, pair, 0)

    cp0.wait()
    cp1.wait()
    prev = None
    for i in range(TPW // 16):
        ga, gb = (gbuf, bbuf) if i % 2 == 0 else (cbuf, dbuf)
        gA = pltpu.async_copy(ys_hbm.at[pbuf[pl.ds(16 * i, 16)]], ga, sem)
        gB = pltpu.async_copy(ys_hbm.at[pbuf[pl.ds(TPW + 16 * i, 16)]], gb, semb)
        gA.wait()
        gB.wait()
        if prev is not None:
            prev.wait()

        def fc(c, _):
            for r in range(16):
                zbuf[r, pl.ds(16 * c, 16)] = (ga[r, pl.ds(16 * c, 16)]
                                              + gb[r, pl.ds(16 * c, 16)])
            return 0
        lax.fori_loop(0, D // 16, fc, 0)
        prev = pltpu.async_copy(zbuf, fin_hbm.at[pl.ds(w * TPW + 16 * i, 16)], semz)
    prev.wait()


@jax.jit
def kernel(x, noise, Wr, br, Wn, bn, W1, b1, W2, b2):
    x2 = x.reshape(S, D)
    noise2 = noise.reshape(S, E)

    gate, e1, e2, g1, g2 = pl.pallas_call(
        _router_body,
        grid=(S // RBLK,),
        in_specs=[
            pl.BlockSpec((RBLK, D), lambda i: (i, 0)),
            pl.BlockSpec((D, E), lambda i: (0, 0)),
            pl.BlockSpec((E,), lambda i: (0,)),
            pl.BlockSpec((D, E), lambda i: (0, 0)),
            pl.BlockSpec((E,), lambda i: (0,)),
            pl.BlockSpec((RBLK, E), lambda i: (i, 0)),
        ],
        out_specs=[
            pl.BlockSpec((RBLK, E), lambda i: (i, 0)),
            pl.BlockSpec((RBLK, 1), lambda i: (i, 0)),
            pl.BlockSpec((RBLK, 1), lambda i: (i, 0)),
            pl.BlockSpec((RBLK, 1), lambda i: (i, 0)),
            pl.BlockSpec((RBLK, 1), lambda i: (i, 0)),
        ],
        out_shape=[
            jax.ShapeDtypeStruct((S, E), jnp.float32),
            jax.ShapeDtypeStruct((S, 1), jnp.int32),
            jax.ShapeDtypeStruct((S, 1), jnp.int32),
            jax.ShapeDtypeStruct((S, 1), jnp.float32),
            jax.ShapeDtypeStruct((S, 1), jnp.float32),
        ],
    )(x2, Wr, br, Wn, bn, noise2)
    e1f, e2f = e1.reshape(S), e2.reshape(S)
    g1f, g2f = g1.reshape(S), g2.reshape(S)

    hist = pl.kernel(
        _hist_body,
        out_type=jax.ShapeDtypeStruct((NW * 16,), jnp.int32),
        mesh=_mesh(),
        compiler_params=pltpu.CompilerParams(needs_layout_passes=False),
        scratch_types=[
            pltpu.VMEM((TPW,), jnp.int32),
            pltpu.VMEM((TPW,), jnp.int32),
            pltpu.VMEM((16,), jnp.int32),
            pltpu.SemaphoreType.DMA,
        ],
    )(e1f, e2f)

    st, sg, xs, p0, p1, bounds, bexp, bval, xsmap = pl.kernel(
        _place_body,
        out_type=(
            jax.ShapeDtypeStruct((ST_ALLOC,), jnp.int32),
            jax.ShapeDtypeStruct((NROWS,), jnp.float32),
            jax.ShapeDtypeStruct((NROWS, D), jnp.float32),
            jax.ShapeDtypeStruct((S,), jnp.int32),
            jax.ShapeDtypeStruct((S,), jnp.int32),
            jax.ShapeDtypeStruct((272,), jnp.int32),
            jax.ShapeDtypeStruct((32,), jnp.int32),
            jax.ShapeDtypeStruct((32,), jnp.int32),
            jax.ShapeDtypeStruct((32,), jnp.int32),
        ),
        mesh=_mesh(),
        compiler_params=pltpu.CompilerParams(needs_layout_passes=False),
        scratch_types=[
            pltpu.VMEM((NW * 16,), jnp.int32),
            pltpu.VMEM((TPW,), jnp.int32),
            pltpu.VMEM((TPW,), jnp.int32),
            pltpu.VMEM((2 * TPW,), jnp.float32),
            pltpu.VMEM((2 * TPW,), jnp.int32),
            pltpu.VMEM((2, TPW), jnp.int32),
            pltpu.VMEM((448,), jnp.int32),
            pltpu.VMEM((TPW, D), jnp.float32),
            pltpu.SemaphoreType.DMA,
            pltpu.SemaphoreType.DMA,
        ],
    )(hist, e1f, e2f, g1f, g2f, x2)

    ys = pl.pallas_call(
        _ffn_body,
        grid_spec=pltpu.PrefetchScalarGridSpec(
            num_scalar_prefetch=3,
            grid=(NBLK,),
            in_specs=[
                pl.BlockSpec((1, T, D), lambda j, be, bv, xm: (xm[j], 0, 0)),
                pl.BlockSpec((1, D, H), lambda j, be, bv, xm: (be[j], 0, 0)),
                pl.BlockSpec((1, 1, H), lambda j, be, bv, xm: (be[j], 0, 0)),
                pl.BlockSpec((1, H, D), lambda j, be, bv, xm: (be[j], 0, 0)),
                pl.BlockSpec((1, 1, D), lambda j, be, bv, xm: (be[j], 0, 0)),
                pl.BlockSpec((1, T, 1), lambda j, be, bv, xm: (xm[j], 0, 0)),
            ],
            out_specs=pl.BlockSpec((1, T, D), lambda j, be, bv, xm: (j, 0, 0)),
        ),
        out_shape=jax.ShapeDtypeStruct((NBLK, T, D), jnp.float32),
        compiler_params=pltpu.CompilerParams(
            dimension_semantics=("arbitrary",),
        ),
    )(bexp, bval, xsmap,
      xs.reshape(NBLK, T, D), W1, b1.reshape(E, 1, H), W2, b2.reshape(E, 1, D),
      sg.reshape(NBLK, T, 1))

    dense, fin = pl.kernel(
        _expand_body,
        out_type=(
            jax.ShapeDtypeStruct((16384 + NDUMP, D), jnp.float32),
            jax.ShapeDtypeStruct((S, D), jnp.float32),
        ),
        mesh=_mesh(),
        compiler_params=pltpu.CompilerParams(needs_layout_passes=False),
        scratch_types=[
            pltpu.VMEM((16, D), jnp.float32),
            pltpu.VMEM((16, D), jnp.float32),
            pltpu.VMEM((16, D), jnp.float32),
            pltpu.VMEM((16, D), jnp.float32),
            pltpu.VMEM((16, D), jnp.float32),
            pltpu.VMEM((TWIN,), jnp.int32),
            pltpu.VMEM((32,), jnp.int32),
            pltpu.VMEM((2 * TPW,), jnp.int32),
            pltpu.SemaphoreType.DMA,
            pltpu.SemaphoreType.DMA,
            pltpu.SemaphoreType.DMA,
        ],
    )(ys.reshape(NROWS, D), st, bounds, p0, p1)

    return (fin.reshape(B, S, D),
            dense[:16384].reshape(E, B, S, D),
            gate.reshape(B, S, E))
